# baseline (speedup 1.0000x reference)
import functools

import jax
import jax.numpy as jnp
from jax import lax
from jax.experimental import pallas as pl
from jax.experimental.pallas import tpu as pltpu
from jax.experimental.pallas import tpu_sc as plsc

B = 16384
F = 26
V = 100000
E = 32
DD = 64
OUT = DD + F * E
FE = F * E

NC, NS, L = 2, 16, 16
NW = NC * NS
PAIRS_W = FE // NW
BH = B // 2
SQ = 4096
V0 = 65536
V1 = V - V0
IDM = (1 << 17) - 1


def _sc_body(sparse_hbm, tables_hbm, inter_hbm,
             er0, er1, spq, pkbuf, col_v, lastf, cnt, semg0, semg1):
    cid = lax.axis_index("c")
    sid = lax.axis_index("s")
    wid = sid * NC + cid
    base = wid * PAIRS_W

    def er0_copy(fe):
        return pltpu.make_async_copy(tables_hbm.at[fe, pl.ds(0, V0)], er0,
                                     semg0)

    def er1_copy(fe):
        return pltpu.make_async_copy(tables_hbm.at[fe, pl.ds(V0, V1)], er1,
                                     semg1)

    lastf[0] = -1
    pltpu.async_copy(tables_hbm.at[base, pl.ds(0, V0)], er0, semg0)

    def pair_body(j, carry):
        fe = base + j
        f = fe // E
        pltpu.async_copy(tables_hbm.at[fe, pl.ds(V0, V1)], er1, semg1)

        @pl.when(f != lastf[0])
        def _partition():
            lastf[0] = f
            for h in range(2):
                def qloop(q, fls):
                    pltpu.sync_copy(
                        sparse_hbm.at[f, pl.ds(h * BH + q * SQ, SQ)], spq)

                    def pgroup(o, fls2):
                        fl2, fb2 = fls2
                        idx = spq[pl.ds(o * L, L)]
                        p = lax.iota(jnp.int32, L) + (q * SQ + o * L)
                        pk = jnp.bitwise_or(idx, jnp.left_shift(p, 17))
                        m = idx < V0
                        plsc.store_compressed(
                            pkbuf.at[pl.ds(h * BH + fl2, L)], pk, mask=m)
                        c = plsc.all_reduce_population_count(m)[0]
                        nfb = fb2 - (L - c)
                        plsc.store_compressed(
                            pkbuf.at[pl.ds(h * BH + nfb, L)], pk,
                            mask=jnp.logical_not(m))
                        return (fl2 + c, nfb)

                    return lax.fori_loop(0, SQ // L, pgroup, fls)

                fl, _ = lax.fori_loop(0, BH // SQ, qloop, (0, BH))
                cnt[h] = fl

        def low_pass(h):
            flg = (cnt[h] // L) * L

            @plsc.parallel_loop(0, flg, L, unroll=16)
            def g_body(g):
                c = pkbuf[pl.ds(h * BH + g, L)]
                x = plsc.load_gather(er0, [jnp.bitwise_and(c, IDM)])
                plsc.store_scatter(col_v, [jnp.right_shift(c, 17)], x)

        def mixed_and_high_pass(h):
            flg = (cnt[h] // L) * L

            @pl.when(flg < BH)
            def _mixed():
                c = pkbuf[pl.ds(h * BH + flg, L)]
                idv = jnp.bitwise_and(c, IDM)
                x0 = plsc.load_gather(er0, [jnp.minimum(idv, V0 - 1)])
                x1 = plsc.load_gather(er1, [jnp.maximum(idv - V0, 0)])
                x = jnp.where(idv < V0, x0, x1)
                plsc.store_scatter(col_v, [jnp.right_shift(c, 17)], x)

            hi0 = flg + L

            @plsc.parallel_loop(0, jnp.maximum(BH - hi0, 0), L, unroll=16)
            def g_body(g):
                c = pkbuf[pl.ds(h * BH + hi0 + g, L)]
                x = plsc.load_gather(er1, [jnp.bitwise_and(c, IDM) - V0])
                plsc.store_scatter(col_v, [jnp.right_shift(c, 17)], x)

        er0_copy(fe).wait()
        low_pass(0)
        er1_copy(fe).wait()
        mixed_and_high_pass(0)
        pltpu.sync_copy(col_v, inter_hbm.at[fe, pl.ds(0, BH)])
        low_pass(1)
        mixed_and_high_pass(1)
        fe_n = jnp.minimum(fe + 1, FE - 1)
        pltpu.async_copy(tables_hbm.at[fe_n, pl.ds(0, V0)], er0, semg0)
        pltpu.sync_copy(col_v, inter_hbm.at[fe, pl.ds(BH, BH)])
        return carry

    lax.fori_loop(0, PAIRS_W, pair_body, 0)
    er0_copy(jnp.minimum(base + PAIRS_W, FE - 1)).wait()


_sc_gather = functools.partial(
    pl.kernel,
    out_type=jax.ShapeDtypeStruct((FE, B), jnp.float32),
    mesh=plsc.VectorSubcoreMesh(core_axis_name="c", subcore_axis_name="s",
                                num_cores=NC, num_subcores=NS),
    compiler_params=pltpu.CompilerParams(needs_layout_passes=False),
    scratch_types=[
        pltpu.VMEM((V0,), jnp.float32),
        pltpu.VMEM((V1,), jnp.float32),
        pltpu.VMEM((SQ,), jnp.int32),
        pltpu.VMEM((B + L,), jnp.int32),
        pltpu.VMEM((BH,), jnp.float32),
        pltpu.SMEM((1,), jnp.int32),
        pltpu.SMEM((2,), jnp.int32),
        pltpu.SemaphoreType.DMA,
        pltpu.SemaphoreType.DMA,
    ],
)(_sc_body)


ROWS_TC = 2048


def _tc_body(dense_ref, inter_ref, out_ref):
    out_ref[:, 0:DD] = dense_ref[...]
    out_ref[:, DD:OUT] = inter_ref[...].T


_tc_combine = functools.partial(
    pl.pallas_call,
    grid=(B // ROWS_TC,),
    in_specs=[
        pl.BlockSpec((ROWS_TC, DD), lambda i: (i, 0)),
        pl.BlockSpec((FE, ROWS_TC), lambda i: (0, i)),
    ],
    out_specs=pl.BlockSpec((ROWS_TC, OUT), lambda i: (i, 0)),
    out_shape=jax.ShapeDtypeStruct((B, OUT), jnp.float32),
)(_tc_body)


@jax.jit
def kernel(dense, sparse, tables):
    tables_t = tables.transpose(0, 2, 1).reshape(FE, V)
    sparse_t = sparse.astype(jnp.int32).T
    inter = _sc_gather(sparse_t, tables_t)
    return _tc_combine(dense, inter)


# speedup vs baseline: 1.0118x; 1.0118x over previous
import functools

import jax
import jax.numpy as jnp
from jax import lax
from jax.experimental import pallas as pl
from jax.experimental.pallas import tpu as pltpu
from jax.experimental.pallas import tpu_sc as plsc

B = 16384
F = 26
V = 100000
E = 32
DD = 64
OUT = DD + F * E
FE = F * E

NC, NS, L = 2, 16, 16
NW = NC * NS
PAIRS_W = FE // NW
BH = B // 2
SQ = 4096
V0 = 65536
V1 = V - V0
IDM = (1 << 17) - 1


def _sc_body(sparse_hbm, tables_hbm, inter_hbm,
             er0, er1, spq, pkbuf, col_v, lastf, cnt, semg0, semg1):
    cid = lax.axis_index("c")
    sid = lax.axis_index("s")
    wid = sid * NC + cid
    base = wid * PAIRS_W

    def er0_copy(fe):
        return pltpu.make_async_copy(tables_hbm.at[fe, pl.ds(0, V0)], er0,
                                     semg0)

    def er1_copy(fe):
        return pltpu.make_async_copy(tables_hbm.at[fe, pl.ds(V0, V1)], er1,
                                     semg1)

    lastf[0] = -1
    pltpu.async_copy(tables_hbm.at[base, pl.ds(0, V0)], er0, semg0)

    def pair_body(j, carry):
        fe = base + j
        f = fe // E
        pltpu.async_copy(tables_hbm.at[fe, pl.ds(V0, V1)], er1, semg1)

        @pl.when(f != lastf[0])
        def _partition():
            lastf[0] = f
            for h in range(2):
                def qloop(q, fls):
                    pltpu.sync_copy(
                        sparse_hbm.at[f, pl.ds(h * BH + q * SQ, SQ)], spq)

                    def pgroup(o, fls2):
                        fl2, fb2 = fls2
                        idx = spq[pl.ds(o * L, L)]
                        p = lax.iota(jnp.int32, L) + (q * SQ + o * L)
                        pk = jnp.bitwise_or(idx, jnp.left_shift(p, 17))
                        m = idx < V0
                        plsc.store_compressed(
                            pkbuf.at[pl.ds(h * BH + fl2, L)], pk, mask=m)
                        c = plsc.all_reduce_population_count(m)[0]
                        nfb = fb2 - (L - c)
                        plsc.store_compressed(
                            pkbuf.at[pl.ds(h * BH + nfb, L)], pk,
                            mask=jnp.logical_not(m))
                        return (fl2 + c, nfb)

                    return lax.fori_loop(0, SQ // L, pgroup, fls)

                fl, _ = lax.fori_loop(0, BH // SQ, qloop, (0, BH))
                cnt[h] = fl

        def low_pass(h):
            flg = (cnt[h] // L) * L

            @plsc.parallel_loop(0, flg, L, unroll=8)
            def g_body(g):
                c = pkbuf[pl.ds(h * BH + g, L)]
                x = plsc.load_gather(er0, [jnp.bitwise_and(c, IDM)])
                plsc.store_scatter(col_v, [jnp.right_shift(c, 17)], x)

        def mixed_and_high_pass(h):
            flg = (cnt[h] // L) * L

            @pl.when(flg < BH)
            def _mixed():
                c = pkbuf[pl.ds(h * BH + flg, L)]
                idv = jnp.bitwise_and(c, IDM)
                x0 = plsc.load_gather(er0, [jnp.minimum(idv, V0 - 1)])
                x1 = plsc.load_gather(er1, [jnp.maximum(idv - V0, 0)])
                x = jnp.where(idv < V0, x0, x1)
                plsc.store_scatter(col_v, [jnp.right_shift(c, 17)], x)

            hi0 = flg + L

            @plsc.parallel_loop(0, jnp.maximum(BH - hi0, 0), L, unroll=8)
            def g_body(g):
                c = pkbuf[pl.ds(h * BH + hi0 + g, L)]
                x = plsc.load_gather(er1, [jnp.bitwise_and(c, IDM) - V0])
                plsc.store_scatter(col_v, [jnp.right_shift(c, 17)], x)

        er0_copy(fe).wait()
        low_pass(0)
        er1_copy(fe).wait()
        mixed_and_high_pass(0)
        pltpu.sync_copy(col_v, inter_hbm.at[fe, pl.ds(0, BH)])
        low_pass(1)
        mixed_and_high_pass(1)
        fe_n = jnp.minimum(fe + 1, FE - 1)
        pltpu.async_copy(tables_hbm.at[fe_n, pl.ds(0, V0)], er0, semg0)
        pltpu.sync_copy(col_v, inter_hbm.at[fe, pl.ds(BH, BH)])
        return carry

    lax.fori_loop(0, PAIRS_W, pair_body, 0)
    er0_copy(jnp.minimum(base + PAIRS_W, FE - 1)).wait()


_sc_gather = functools.partial(
    pl.kernel,
    out_type=jax.ShapeDtypeStruct((FE, B), jnp.float32),
    mesh=plsc.VectorSubcoreMesh(core_axis_name="c", subcore_axis_name="s",
                                num_cores=NC, num_subcores=NS),
    compiler_params=pltpu.CompilerParams(needs_layout_passes=False),
    scratch_types=[
        pltpu.VMEM((V0,), jnp.float32),
        pltpu.VMEM((V1,), jnp.float32),
        pltpu.VMEM((SQ,), jnp.int32),
        pltpu.VMEM((B + L,), jnp.int32),
        pltpu.VMEM((BH,), jnp.float32),
        pltpu.SMEM((1,), jnp.int32),
        pltpu.SMEM((2,), jnp.int32),
        pltpu.SemaphoreType.DMA,
        pltpu.SemaphoreType.DMA,
    ],
)(_sc_body)


ROWS_TC = 2048


def _tc_body(dense_ref, inter_ref, out_ref):
    out_ref[:, 0:DD] = dense_ref[...]
    out_ref[:, DD:OUT] = inter_ref[...].T


_tc_combine = functools.partial(
    pl.pallas_call,
    grid=(B // ROWS_TC,),
    in_specs=[
        pl.BlockSpec((ROWS_TC, DD), lambda i: (i, 0)),
        pl.BlockSpec((FE, ROWS_TC), lambda i: (0, i)),
    ],
    out_specs=pl.BlockSpec((ROWS_TC, OUT), lambda i: (i, 0)),
    out_shape=jax.ShapeDtypeStruct((B, OUT), jnp.float32),
)(_tc_body)


@jax.jit
def kernel(dense, sparse, tables):
    tables_t = tables.transpose(0, 2, 1).reshape(FE, V)
    sparse_t = sparse.astype(jnp.int32).T
    inter = _sc_gather(sparse_t, tables_t)
    return _tc_combine(dense, inter)
